# baseline (speedup 1.0000x reference)
import functools

import jax
import jax.numpy as jnp
from jax import lax
from jax.experimental import pallas as pl
from jax.experimental.pallas import tpu as pltpu
from jax.experimental.pallas import tpu_sc as plsc

MAX_LEN = 2048
NUM_HEADS = 16
RB = 16
NU = 8
LM = 4224
LPAD = 136
WIN = 2944
NSC = 4

_info = plsc.get_sparse_core_info()
_NC, _NS = _info.num_cores, _info.num_subcores
_NW = _NC * _NS
_ROWS_PER = (NUM_HEADS * MAX_LEN) // _NW
_WPH = MAX_LEN // _ROWS_PER


def _make_sc_kernel():
    mesh = plsc.VectorSubcoreMesh(core_axis_name="c", subcore_axis_name="s")

    @functools.partial(
        pl.kernel,
        mesh=mesh,
        out_type=jax.ShapeDtypeStruct((NUM_HEADS, MAX_LEN, MAX_LEN), jnp.float32),
        scratch_types=[pltpu.VMEM((RB, WIN), jnp.float32)] * 2
        + [pltpu.SemaphoreType.DMA] * (2 * NSC + 2),
    )
    def sc_bias(l3_hbm, out_hbm, *scratch):
        win = scratch[:2]
        sems = scratch[2 : 2 + 2 * NSC]
        ssem = scratch[2 + 2 * NSC :]
        wid = lax.axis_index("s") * _NC + lax.axis_index("c")
        h = wid // _WPH
        p = wid % _WPH

        mbase = pl.multiple_of(1152 - 1024 * p, 128)

        def stage_copy(u, g):
            return pltpu.make_async_copy(
                l3_hbm.at[u * NUM_HEADS + h, :, pl.ds(mbase, WIN)], win[g], ssem[g]
            )

        def stage_wait(g):
            pltpu.make_async_copy(
                l3_hbm.at[0, :, pl.ds(0, WIN)], win[g], ssem[g]
            ).wait()

        def scatter(u, g, n8):
            row0 = pl.multiple_of(RB * (u + 64 * p + 8 * n8), 8)
            return pltpu.make_async_copy(
                win[g].at[:, pl.ds(128 * (7 - n8), MAX_LEN)],
                out_hbm.at[h, pl.ds(row0, RB), :],
                sems[NSC * g + (n8 % NSC)],
            )

        def scatter_wait(g, slot):
            pltpu.make_async_copy(
                win[g].at[:, pl.ds(0, MAX_LEN)],
                out_hbm.at[h, pl.ds(0, RB), :],
                sems[NSC * g + slot],
            ).wait()

        stage_copy(0, 0).start()

        def blk(u2, carry):
            for g in range(2):
                u = 2 * u2 + g
                stage_wait(g)
                for n8 in range(NSC):
                    scatter(u, g, n8).start()
                if g == 1:
                    for n8 in range(NSC, 8):
                        scatter_wait(1 - g, n8 % NSC)
                    @pl.when(u2 < NU // 2 - 1)
                    def _stage_next():
                        stage_copy(u + 1, 1 - g).start()
                else:
                    @pl.when(u2 > 0)
                    def _drain_prev():
                        for n8 in range(NSC, 8):
                            scatter_wait(1 - g, n8 % NSC)

                    stage_copy(u + 1, 1 - g).start()

                for n8 in range(NSC, 8):
                    scatter_wait(g, n8 % NSC)
                    scatter(u, g, n8).start()
            return carry

        lax.fori_loop(0, NU // 2, blk, 0)
        for n8 in range(NSC, 8):
            scatter_wait(1, n8 % NSC)

    return sc_bias


_sc_bias = _make_sc_kernel()


@jax.jit
def kernel(T, table):
    del T
    ttp = jnp.pad(jnp.transpose(table), ((0, 0), (LPAD, 129)))
    l4 = jnp.stack(
        [
            jnp.stack(
                [
                    ttp[:, LPAD - si - (RB * u + 1) : LPAD - si - (RB * u + 1) + LM]
                    for si in range(RB)
                ],
                axis=1,
            )
            for u in range(NU)
        ],
        axis=0,
    )
    rt_zero = jnp.minimum(jnp.abs(table[0, 0]), jnp.float32(0.0))
    return _sc_bias(l4.reshape(NU * NUM_HEADS, RB, LM) + rt_zero)


# speedup vs baseline: 1.1167x; 1.1167x over previous
import functools

import jax
import jax.numpy as jnp
from jax import lax
from jax.experimental import pallas as pl
from jax.experimental.pallas import tpu as pltpu
from jax.experimental.pallas import tpu_sc as plsc

MAX_LEN = 2048
NUM_HEADS = 16
RB = 16
NU = 8
LM = 4224
LPAD = 136
WIN = 2944
NSC = 4

_info = plsc.get_sparse_core_info()
_NC, _NS = _info.num_cores, _info.num_subcores
_NW = _NC * _NS
_ROWS_PER = (NUM_HEADS * MAX_LEN) // _NW
_WPH = MAX_LEN // _ROWS_PER


def _make_sc_kernel():
    mesh = plsc.VectorSubcoreMesh(core_axis_name="c", subcore_axis_name="s")

    @functools.partial(
        pl.kernel,
        mesh=mesh,
        out_type=jax.ShapeDtypeStruct((NUM_HEADS, MAX_LEN, MAX_LEN), jnp.float32),
        scratch_types=[pltpu.VMEM((RB, WIN), jnp.float32)] * 2
        + [pltpu.SemaphoreType.DMA] * (2 * NSC + 2),
    )
    def sc_bias(l3_hbm, out_hbm, *scratch):
        win = scratch[:2]
        sems = scratch[2 : 2 + 2 * NSC]
        ssem = scratch[2 + 2 * NSC :]
        wid = lax.axis_index("s") * _NC + lax.axis_index("c")
        h = wid // _WPH
        p = wid % _WPH

        mbase = pl.multiple_of(1152 - 1024 * p, 128)

        def stage_copy(u, g):
            return pltpu.make_async_copy(
                l3_hbm.at[u * NUM_HEADS + h, :, pl.ds(mbase, WIN)], win[g], ssem[g]
            )

        def stage_wait(g):
            pltpu.make_async_copy(
                l3_hbm.at[0, :, pl.ds(0, WIN)], win[g], ssem[g]
            ).wait()

        def scatter(u, g, n8):
            row0 = pl.multiple_of(RB * (u + 64 * p + 8 * n8), 8)
            return pltpu.make_async_copy(
                win[g].at[:, pl.ds(128 * (7 - n8), MAX_LEN)],
                out_hbm.at[h, pl.ds(row0, RB), :],
                sems[NSC * g + (n8 % NSC)],
            )

        def scatter_wait(g, slot):
            pltpu.make_async_copy(
                win[g].at[:, pl.ds(0, MAX_LEN)],
                out_hbm.at[h, pl.ds(0, RB), :],
                sems[NSC * g + slot],
            ).wait()

        stage_copy(0, 0).start()

        def blk(u2, carry):
            for g in range(2):
                u = 2 * u2 + g
                stage_wait(g)
                for n8 in range(NSC):
                    scatter(u, g, n8).start()
                if g == 1:
                    for n8 in range(NSC, 8):
                        scatter_wait(1 - g, n8 % NSC)
                    @pl.when(u2 < NU // 2 - 1)
                    def _stage_next():
                        stage_copy(u + 1, 1 - g).start()
                else:
                    @pl.when(u2 > 0)
                    def _drain_prev():
                        for n8 in range(NSC, 8):
                            scatter_wait(1 - g, n8 % NSC)

                    stage_copy(u + 1, 1 - g).start()

                for n8 in range(NSC, 8):
                    scatter_wait(g, n8 % NSC)
                    scatter(u, g, n8).start()
            return carry

        lax.fori_loop(0, NU // 2, blk, 0)
        for n8 in range(NSC, 8):
            scatter_wait(1, n8 % NSC)

    return sc_bias


_sc_bias = _make_sc_kernel()


@jax.jit
def kernel(T, table):
    del T
    ttp = jnp.pad(jnp.transpose(table), ((0, 0), (LPAD, 129)))
    l4 = jnp.stack(
        [
            jnp.stack(
                [
                    ttp[:, LPAD - si - (RB * u + 1) : LPAD - si - (RB * u + 1) + LM]
                    for si in range(RB)
                ],
                axis=1,
            )
            for u in range(NU)
        ],
        axis=0,
    )
    return _sc_bias(l4.reshape(NU * NUM_HEADS, RB, LM))
